# baseline (speedup 1.0000x reference)
import functools

import jax
import jax.numpy as jnp
from jax import lax
from jax.experimental import pallas as pl
from jax.experimental.pallas import tpu as pltpu, tpu_sc as plsc

B, N, D, K = 4, 4096, 64, 1024
DP = 128
TILE_N = 4096
NT = N // TILE_N

NC, NS = 2, 16
NW = NC * NS
TOK = B * N
CHUNK = TOK // NW


def _tc1_kernel(enc_ref, dec_ref, cbt_ref, wpre_ref, bpre_ref, wpost_ref,
                cb_ref, idx_ref, loss_ref, cbw_ref):
    b = pl.program_id(0)

    enc = enc_ref[0]
    dec = dec_ref[0]
    cbt = cbt_ref[...]

    res = jnp.dot(enc - dec, wpre_ref[...],
                  preferred_element_type=jnp.float32) + bpre_ref[...]

    cb_norm = jnp.sum(cbt * cbt, axis=0).reshape(1, K)
    scores = jnp.dot(res, cbt, preferred_element_type=jnp.float32)
    d2 = cb_norm - 2.0 * scores

    md = jnp.min(d2, axis=1).reshape(TILE_N, 1)
    iota = jax.lax.broadcasted_iota(jnp.int32, (TILE_N, K), 1)
    masked = jnp.where(d2 <= md, iota, K)
    idx_ref[0] = jnp.min(masked, axis=1).reshape(TILE_N, 1)

    part = jnp.sum(md) + jnp.sum(res * res)
    loss_ref[...] = (part * (1.0 / (N * D))).reshape(1, 1, 1)

    @pl.when(b == 0)
    def _cbw():
        cbw = jnp.dot(cb_ref[...], wpost_ref[...],
                      preferred_element_type=jnp.float32)
        cbw_ref[...] = jnp.pad(cbw, ((0, 0), (0, DP - D)))


def _sc_kernel(cbw_hbm, idx_hbm, quant_hbm, counts_hbm,
               idx_v, rows_v, hist_v, sem):
    wid = lax.axis_index("s") * NC + lax.axis_index("c")
    base = wid * CHUNK

    pltpu.sync_copy(idx_hbm.at[pl.ds(base, CHUNK)], idx_v)
    pltpu.async_copy(cbw_hbm.at[idx_v], rows_v, sem).wait()
    pltpu.sync_copy(rows_v, quant_hbm.at[pl.ds(base, CHUNK)])

    zeros = jnp.zeros((16,), jnp.float32)
    for i in range(K // 16):
        hist_v[pl.ds(i * 16, 16)] = zeros
    ones = jnp.ones((16,), jnp.float32)
    for i in range(CHUNK // 16):
        plsc.addupdate_scatter(hist_v, [idx_v[pl.ds(i * 16, 16)]], ones)
    pltpu.sync_copy(hist_v, counts_hbm.at[wid])


def _tc2_kernel(quant_ref, dec_ref, wpost_ref, bpost_ref, counts_ref,
                out_ref, kl_ref):
    out_ref[0] = (quant_ref[0, :, :D]
                  + jnp.dot(dec_ref[0], wpost_ref[...],
                            preferred_element_type=jnp.float32)
                  + bpost_ref[...])

    counts = jnp.sum(counts_ref[...], axis=0)
    probs = counts.reshape(1, K) * (1.0 / N)
    kl = jnp.sum(probs * jnp.log(probs * K + 1e-10))
    kl_ref[...] = kl.reshape(1, 1, 1)


@jax.jit
def kernel(enc, dec, codebook, W_pre, b_pre, W_post, b_post):
    idx, loss, cbw = pl.pallas_call(
        _tc1_kernel,
        grid=(B,),
        in_specs=[
            pl.BlockSpec((1, TILE_N, D), lambda b: (b, 0, 0)),
            pl.BlockSpec((1, TILE_N, D), lambda b: (b, 0, 0)),
            pl.BlockSpec((D, K), lambda b: (0, 0)),
            pl.BlockSpec((D, D), lambda b: (0, 0)),
            pl.BlockSpec((1, D), lambda b: (0, 0)),
            pl.BlockSpec((D, D), lambda b: (0, 0)),
            pl.BlockSpec((K, D), lambda b: (0, 0)),
        ],
        out_specs=[
            pl.BlockSpec((1, TILE_N, 1), lambda b: (b, 0, 0)),
            pl.BlockSpec((1, 1, 1), lambda b: (b, 0, 0)),
            pl.BlockSpec((K, DP), lambda b: (0, 0)),
        ],
        out_shape=[
            jax.ShapeDtypeStruct((B, N, 1), jnp.int32),
            jax.ShapeDtypeStruct((B, 1, 1), jnp.float32),
            jax.ShapeDtypeStruct((K, DP), jnp.float32),
        ],
    )(enc, dec, codebook.T, W_pre, b_pre.reshape(1, D), W_post, codebook)

    mesh = plsc.VectorSubcoreMesh(core_axis_name="c", subcore_axis_name="s")
    quant_flat, counts32 = pl.kernel(
        _sc_kernel,
        mesh=mesh,
        out_type=[
            jax.ShapeDtypeStruct((TOK, DP), jnp.float32),
            jax.ShapeDtypeStruct((NW, K), jnp.float32),
        ],
        scratch_types=[
            pltpu.VMEM((CHUNK,), jnp.int32),
            pltpu.VMEM((CHUNK, DP), jnp.float32),
            pltpu.VMEM((K,), jnp.float32),
            pltpu.SemaphoreType.DMA,
        ],
        compiler_params=pltpu.CompilerParams(needs_layout_passes=False),
    )(cbw, idx.reshape(TOK))

    quantw = quant_flat.reshape(B, N, DP)
    out, kl = pl.pallas_call(
        _tc2_kernel,
        grid=(B,),
        in_specs=[
            pl.BlockSpec((1, TILE_N, DP), lambda b: (b, 0, 0)),
            pl.BlockSpec((1, TILE_N, D), lambda b: (b, 0, 0)),
            pl.BlockSpec((D, D), lambda b: (0, 0)),
            pl.BlockSpec((1, D), lambda b: (0, 0)),
            pl.BlockSpec((8, K), lambda b: (b, 0)),
        ],
        out_specs=[
            pl.BlockSpec((1, TILE_N, D), lambda b: (b, 0, 0)),
            pl.BlockSpec((1, 1, 1), lambda b: (b, 0, 0)),
        ],
        out_shape=[
            jax.ShapeDtypeStruct((B, N, D), jnp.float32),
            jax.ShapeDtypeStruct((B, 1, 1), jnp.float32),
        ],
    )(quantw, dec, W_post, b_post.reshape(1, D), counts32)

    loss = loss.reshape(B)
    kl = kl.reshape(B)
    return out, loss, loss, kl


# speedup vs baseline: 1.1523x; 1.1523x over previous
import jax
import jax.numpy as jnp
from jax.experimental import pallas as pl
from jax.experimental.pallas import tpu as pltpu

B, N, D, K = 4, 4096, 64, 1024


def _fused_kernel(enc_ref, dec_ref, cbt_ref, wpre_ref, bpre_ref, wpost_ref,
                  bpost_ref, cb_ref, out_ref, loss_ref, kl_ref,
                  cbw_hi_ref, cbw_lo_ref):
    b = pl.program_id(0)

    enc = enc_ref[0]
    dec = dec_ref[0]
    cbt = cbt_ref[...]

    @pl.when(b == 0)
    def _cbw():
        cbw = jnp.dot(cb_ref[...], wpost_ref[...],
                      preferred_element_type=jnp.float32)
        hi = cbw.astype(jnp.bfloat16)
        cbw_hi_ref[...] = hi
        cbw_lo_ref[...] = (cbw - hi.astype(jnp.float32)).astype(jnp.bfloat16)

    res = jnp.dot(enc - dec, wpre_ref[...],
                  preferred_element_type=jnp.float32) + bpre_ref[...]

    cb_norm = jnp.sum(cbt * cbt, axis=0).reshape(1, K)
    scores = jnp.dot(res, cbt, preferred_element_type=jnp.float32)
    d2 = cb_norm - 2.0 * scores

    md = jnp.min(d2, axis=1).reshape(N, 1)
    iota = jax.lax.broadcasted_iota(jnp.int32, (N, K), 1)
    masked = jnp.where(d2 <= md, iota, K)
    idx = jnp.min(masked, axis=1).reshape(N, 1)
    onehot = (iota == idx).astype(jnp.bfloat16)

    counts = jnp.sum(onehot.astype(jnp.float32), axis=0).reshape(1, K)
    probs = counts * (1.0 / N)
    kl = jnp.sum(probs * jnp.log(probs * K + 1e-10))
    kl_ref[...] = kl.reshape(1, 1, 1)

    part = jnp.sum(md) + jnp.sum(res * res)
    loss_ref[...] = (part * (1.0 / (N * D))).reshape(1, 1, 1)

    quant_w = (jnp.dot(onehot, cbw_hi_ref[...],
                       preferred_element_type=jnp.float32)
               + jnp.dot(onehot, cbw_lo_ref[...],
                         preferred_element_type=jnp.float32))
    out_ref[0] = (quant_w
                  + jnp.dot(dec, wpost_ref[...],
                            preferred_element_type=jnp.float32)
                  + bpost_ref[...])


@jax.jit
def kernel(enc, dec, codebook, W_pre, b_pre, W_post, b_post):
    out, loss, kl = pl.pallas_call(
        _fused_kernel,
        grid=(B,),
        in_specs=[
            pl.BlockSpec((1, N, D), lambda b: (b, 0, 0)),
            pl.BlockSpec((1, N, D), lambda b: (b, 0, 0)),
            pl.BlockSpec((D, K), lambda b: (0, 0)),
            pl.BlockSpec((D, D), lambda b: (0, 0)),
            pl.BlockSpec((1, D), lambda b: (0, 0)),
            pl.BlockSpec((D, D), lambda b: (0, 0)),
            pl.BlockSpec((1, D), lambda b: (0, 0)),
            pl.BlockSpec((K, D), lambda b: (0, 0)),
        ],
        out_specs=[
            pl.BlockSpec((1, N, D), lambda b: (b, 0, 0)),
            pl.BlockSpec((1, 1, 1), lambda b: (b, 0, 0)),
            pl.BlockSpec((1, 1, 1), lambda b: (b, 0, 0)),
        ],
        out_shape=[
            jax.ShapeDtypeStruct((B, N, D), jnp.float32),
            jax.ShapeDtypeStruct((B, 1, 1), jnp.float32),
            jax.ShapeDtypeStruct((B, 1, 1), jnp.float32),
        ],
        scratch_shapes=[
            pltpu.VMEM((K, D), jnp.bfloat16),
            pltpu.VMEM((K, D), jnp.bfloat16),
        ],
    )(enc, dec, codebook.T, W_pre, b_pre.reshape(1, D), W_post,
      b_post.reshape(1, D), codebook)
    loss = loss.reshape(B)
    kl = kl.reshape(B)
    return out, loss, loss, kl


# speedup vs baseline: 1.4794x; 1.2839x over previous
import jax
import jax.numpy as jnp
from jax.experimental import pallas as pl

B, N, D, K = 4, 4096, 64, 1024
TILE_N = 4096
NT = N // TILE_N


def _fused_kernel(enc_ref, dec_ref, cb_ref, cbt_ref, wpre_ref, bpre_ref,
                  wpost_ref, bpost_ref, out_ref, loss_ref, kl_ref,
                  counts_ref):
    n = pl.program_id(1)

    enc = enc_ref[0]
    dec = dec_ref[0]
    cb = cb_ref[...]
    cbt = cbt_ref[...]

    res = jnp.dot(enc - dec, wpre_ref[...],
                  preferred_element_type=jnp.float32) + bpre_ref[...]

    cb_norm = jnp.sum(cbt * cbt, axis=0).reshape(1, K)
    scores = jnp.dot(res, cbt, preferred_element_type=jnp.float32)
    d2 = cb_norm - 2.0 * scores

    idx = jnp.argmin(d2, axis=1).reshape(TILE_N, 1)
    onehot = (jax.lax.broadcasted_iota(jnp.int32, (TILE_N, K), 1)
              == idx).astype(jnp.float32)

    quant = jnp.dot(onehot, cb, preferred_element_type=jnp.float32)

    part_counts = jnp.sum(onehot, axis=0).reshape(1, 1, K)
    diff = quant - res
    part_loss = jnp.sum(diff * diff).reshape(1, 1, 1)

    @pl.when(n == 0)
    def _init():
        counts_ref[...] = part_counts
        loss_ref[...] = part_loss

    @pl.when(n != 0)
    def _acc():
        counts_ref[...] = counts_ref[...] + part_counts
        loss_ref[...] = loss_ref[...] + part_loss

    @pl.when(n == NT - 1)
    def _finalize():
        loss_ref[...] = loss_ref[...] * (1.0 / (N * D))
        probs = counts_ref[...] * (1.0 / N)
        kl = jnp.sum(probs * jnp.log(probs * K + 1e-10))
        kl_ref[...] = kl.reshape(1, 1, 1)

    out_ref[0] = jnp.dot(quant + dec, wpost_ref[...],
                         preferred_element_type=jnp.float32) + bpost_ref[...]


@jax.jit
def kernel(enc, dec, codebook, W_pre, b_pre, W_post, b_post):
    out, loss, kl, _ = pl.pallas_call(
        _fused_kernel,
        grid=(B, NT),
        in_specs=[
            pl.BlockSpec((1, TILE_N, D), lambda b, n: (b, n, 0)),
            pl.BlockSpec((1, TILE_N, D), lambda b, n: (b, n, 0)),
            pl.BlockSpec((K, D), lambda b, n: (0, 0)),
            pl.BlockSpec((D, K), lambda b, n: (0, 0)),
            pl.BlockSpec((D, D), lambda b, n: (0, 0)),
            pl.BlockSpec((1, D), lambda b, n: (0, 0)),
            pl.BlockSpec((D, D), lambda b, n: (0, 0)),
            pl.BlockSpec((1, D), lambda b, n: (0, 0)),
        ],
        out_specs=[
            pl.BlockSpec((1, TILE_N, D), lambda b, n: (b, n, 0)),
            pl.BlockSpec((1, 1, 1), lambda b, n: (b, 0, 0)),
            pl.BlockSpec((1, 1, 1), lambda b, n: (b, 0, 0)),
            pl.BlockSpec((1, 1, K), lambda b, n: (b, 0, 0)),
        ],
        out_shape=[
            jax.ShapeDtypeStruct((B, N, D), jnp.float32),
            jax.ShapeDtypeStruct((B, 1, 1), jnp.float32),
            jax.ShapeDtypeStruct((B, 1, 1), jnp.float32),
            jax.ShapeDtypeStruct((B, 1, K), jnp.float32),
        ],
    )(enc, dec, codebook, codebook.T, W_pre, b_pre.reshape(1, D), W_post,
      b_post.reshape(1, D))
    loss = loss.reshape(B)
    kl = kl.reshape(B)
    return out, loss, loss, kl


# speedup vs baseline: 1.6574x; 1.1203x over previous
import jax
import jax.numpy as jnp
from jax.experimental import pallas as pl

B, N, D, K = 4, 4096, 64, 1024
H = 2
HN = N // H


def _fused_kernel(enc_ref, dec_ref, cb_ref, cbt_ref, wpre_ref, bpre_ref,
                  wpost_ref, bpost_ref, out_ref, loss_ref, kl_ref,
                  counts_ref):
    cb = cb_ref[...]
    cbt = cbt_ref[...]
    cb_norm = jnp.sum(cbt * cbt, axis=0).reshape(1, K)
    iota = jax.lax.broadcasted_iota(jnp.int32, (HN, K), 1)

    quants, parts_counts, parts_loss = [], [], []
    for h in range(H):
        enc = enc_ref[0, pl.ds(h * HN, HN), :]
        dec = dec_ref[0, pl.ds(h * HN, HN), :]
        res = jnp.dot(enc - dec, wpre_ref[...],
                      preferred_element_type=jnp.float32) + bpre_ref[...]
        scores = jnp.dot(res, cbt, preferred_element_type=jnp.float32)
        d2 = cb_norm - 2.0 * scores
        idx = jnp.argmin(d2, axis=1).reshape(HN, 1)
        onehot = (iota == idx).astype(jnp.float32)
        quant = jnp.dot(onehot, cb, preferred_element_type=jnp.float32)
        diff = quant - res
        quants.append(quant + dec)
        parts_counts.append(jnp.sum(onehot, axis=0).reshape(1, 1, K))
        parts_loss.append(jnp.sum(diff * diff))

    counts = parts_counts[0] + parts_counts[1]
    counts_ref[...] = counts
    probs = counts * (1.0 / N)
    kl = jnp.sum(probs * jnp.log(probs * K + 1e-10))
    kl_ref[...] = kl.reshape(1, 1, 1)
    loss_ref[...] = ((parts_loss[0] + parts_loss[1])
                     * (1.0 / (N * D))).reshape(1, 1, 1)

    qd = jnp.concatenate(quants, axis=0)
    out_ref[0] = jnp.dot(qd, wpost_ref[...],
                         preferred_element_type=jnp.float32) + bpost_ref[...]


@jax.jit
def kernel(enc, dec, codebook, W_pre, b_pre, W_post, b_post):
    out, loss, kl, _ = pl.pallas_call(
        _fused_kernel,
        grid=(B,),
        in_specs=[
            pl.BlockSpec((1, N, D), lambda b: (b, 0, 0)),
            pl.BlockSpec((1, N, D), lambda b: (b, 0, 0)),
            pl.BlockSpec((K, D), lambda b: (0, 0)),
            pl.BlockSpec((D, K), lambda b: (0, 0)),
            pl.BlockSpec((D, D), lambda b: (0, 0)),
            pl.BlockSpec((1, D), lambda b: (0, 0)),
            pl.BlockSpec((D, D), lambda b: (0, 0)),
            pl.BlockSpec((1, D), lambda b: (0, 0)),
        ],
        out_specs=[
            pl.BlockSpec((1, N, D), lambda b: (b, 0, 0)),
            pl.BlockSpec((1, 1, 1), lambda b: (b, 0, 0)),
            pl.BlockSpec((1, 1, 1), lambda b: (b, 0, 0)),
            pl.BlockSpec((1, 1, K), lambda b: (b, 0, 0)),
        ],
        out_shape=[
            jax.ShapeDtypeStruct((B, N, D), jnp.float32),
            jax.ShapeDtypeStruct((B, 1, 1), jnp.float32),
            jax.ShapeDtypeStruct((B, 1, 1), jnp.float32),
            jax.ShapeDtypeStruct((B, 1, K), jnp.float32),
        ],
    )(enc, dec, codebook, codebook.T, W_pre, b_pre.reshape(1, D), W_post,
      b_post.reshape(1, D))
    loss = loss.reshape(B)
    kl = kl.reshape(B)
    return out, loss, loss, kl


# speedup vs baseline: 1.7165x; 1.0357x over previous
import jax
import jax.numpy as jnp
from jax.experimental import pallas as pl

B, N, D, K = 4, 4096, 64, 1024
H = 4
HN = N // H


def _fused_kernel(enc_ref, dec_ref, cb_ref, cbt_ref, wpre_ref, bpre_ref,
                  wpost_ref, bpost_ref, out_ref, loss_ref, kl_ref,
                  counts_ref):
    cb = cb_ref[...]
    cbt = cbt_ref[...]
    cb_norm = jnp.sum(cbt * cbt, axis=0).reshape(1, K)
    iota = jax.lax.broadcasted_iota(jnp.int32, (HN, K), 1)

    quants, parts_counts, parts_loss = [], [], []
    for h in range(H):
        enc = enc_ref[0, pl.ds(h * HN, HN), :]
        dec = dec_ref[0, pl.ds(h * HN, HN), :]
        res = jnp.dot(enc - dec, wpre_ref[...],
                      preferred_element_type=jnp.float32) + bpre_ref[...]
        scores = jnp.dot(res, cbt, preferred_element_type=jnp.float32)
        d2 = cb_norm - 2.0 * scores
        idx = jnp.argmin(d2, axis=1).reshape(HN, 1)
        onehot = (iota == idx).astype(jnp.float32)
        quant = jnp.dot(onehot, cb, preferred_element_type=jnp.float32)
        diff = quant - res
        quants.append(quant + dec)
        parts_counts.append(jnp.sum(onehot, axis=0).reshape(1, 1, K))
        parts_loss.append(jnp.sum(diff * diff))

    counts = sum(parts_counts)
    counts_ref[...] = counts
    probs = counts * (1.0 / N)
    kl = jnp.sum(probs * jnp.log(probs * K + 1e-10))
    kl_ref[...] = kl.reshape(1, 1, 1)
    loss_ref[...] = (sum(parts_loss)
                     * (1.0 / (N * D))).reshape(1, 1, 1)

    qd = jnp.concatenate(quants, axis=0)
    out_ref[0] = jnp.dot(qd, wpost_ref[...],
                         preferred_element_type=jnp.float32) + bpost_ref[...]


@jax.jit
def kernel(enc, dec, codebook, W_pre, b_pre, W_post, b_post):
    out, loss, kl, _ = pl.pallas_call(
        _fused_kernel,
        grid=(B,),
        in_specs=[
            pl.BlockSpec((1, N, D), lambda b: (b, 0, 0)),
            pl.BlockSpec((1, N, D), lambda b: (b, 0, 0)),
            pl.BlockSpec((K, D), lambda b: (0, 0)),
            pl.BlockSpec((D, K), lambda b: (0, 0)),
            pl.BlockSpec((D, D), lambda b: (0, 0)),
            pl.BlockSpec((1, D), lambda b: (0, 0)),
            pl.BlockSpec((D, D), lambda b: (0, 0)),
            pl.BlockSpec((1, D), lambda b: (0, 0)),
        ],
        out_specs=[
            pl.BlockSpec((1, N, D), lambda b: (b, 0, 0)),
            pl.BlockSpec((1, 1, 1), lambda b: (b, 0, 0)),
            pl.BlockSpec((1, 1, 1), lambda b: (b, 0, 0)),
            pl.BlockSpec((1, 1, K), lambda b: (b, 0, 0)),
        ],
        out_shape=[
            jax.ShapeDtypeStruct((B, N, D), jnp.float32),
            jax.ShapeDtypeStruct((B, 1, 1), jnp.float32),
            jax.ShapeDtypeStruct((B, 1, 1), jnp.float32),
            jax.ShapeDtypeStruct((B, 1, K), jnp.float32),
        ],
    )(enc, dec, codebook, codebook.T, W_pre, b_pre.reshape(1, D), W_post,
      b_post.reshape(1, D))
    loss = loss.reshape(B)
    kl = kl.reshape(B)
    return out, loss, loss, kl
